# Initial kernel scaffold; baseline (speedup 1.0000x reference)
import functools

import jax
import jax.numpy as jnp
from jax import lax
from jax.experimental import pallas as pl
from jax.experimental.pallas import tpu as pltpu
from jax.experimental.pallas import tpu_sc as plsc

NU = 50000
NI = 50000
N = NU + NI
E = 3_200_000
D = 16
B = 16384
N_LAYERS = 3

NC = 2
NS = 16
NW = NC * NS

CHUNK_ROWS = 16
LANES = 128
CHUNK = CHUNK_ROWS * LANES
ITERS = 50
EDGES_PER_WORKER = CHUNK * ITERS
EP = EDGES_PER_WORKER * NW
ROWS_PER_WORKER = EDGES_PER_WORKER // LANES

NODES_PER_TILE = N // NS
ZROWS = 1250

_mesh = plsc.VectorSubcoreMesh(
    core_axis_name="c", subcore_axis_name="s", num_cores=NC, num_subcores=NS
)


@functools.partial(
    pl.kernel,
    out_type=jax.ShapeDtypeStruct((NC, N, D), jnp.float32),
    mesh=_mesh,
    scratch_types=[
        pltpu.VMEM((CHUNK_ROWS, LANES), jnp.int32),
        pltpu.VMEM((CHUNK_ROWS, LANES), jnp.int32),
        pltpu.VMEM((CHUNK_ROWS, LANES), jnp.float32),
        pltpu.VMEM((CHUNK_ROWS, LANES, D), jnp.float32),
        pltpu.VMEM((ZROWS, D), jnp.float32),
        pltpu.VMEM_SHARED((N, D), jnp.float32),
        pltpu.SemaphoreType.DMA,
    ],
)
def _propagate(src_hbm, dst_hbm, vals_hbm, table_hbm, out_hbm,
               sidx, didx, vref, rows, zbuf, acc, sem):
    cid = lax.axis_index("c")
    sid = lax.axis_index("s")
    wid = sid * NC + cid

    zbuf[...] = jnp.zeros((ZROWS, D), jnp.float32)
    for z in range(NODES_PER_TILE // ZROWS):
        pltpu.sync_copy(
            zbuf, acc.at[pl.ds(sid * NODES_PER_TILE + z * ZROWS, ZROWS)]
        )
    plsc.subcore_barrier()

    row_base = wid * ROWS_PER_WORKER

    def body(it, _):
        r0 = row_base + it * CHUNK_ROWS
        pltpu.sync_copy(src_hbm.at[pl.ds(r0, CHUNK_ROWS)], sidx)
        pltpu.sync_copy(dst_hbm.at[pl.ds(r0, CHUNK_ROWS)], didx)
        pltpu.sync_copy(vals_hbm.at[pl.ds(r0, CHUNK_ROWS)], vref)
        pltpu.async_copy(table_hbm.at[sidx], rows, sem).wait()

        def scale(e, _):
            r = e >> 7
            c = e & 127
            rows[r, c, :] = rows[r, c, :] * vref[r, c]
            return ()

        lax.fori_loop(0, CHUNK, scale, (), unroll=8)
        pltpu.sync_copy(rows, acc.at[didx], add=True)
        return ()

    lax.fori_loop(0, ITERS, body, ())

    plsc.subcore_barrier()
    lo = sid * NODES_PER_TILE
    pltpu.sync_copy(
        acc.at[pl.ds(lo, NODES_PER_TILE)],
        out_hbm.at[cid].at[pl.ds(lo, NODES_PER_TILE)],
    )


def _combine_body(p0_ref, p1_ref, sum_ref, table_out, sum_out):
    t = p0_ref[...] + p1_ref[...]
    table_out[...] = t
    sum_out[...] = sum_ref[...] + t


def _combine(p0, p1, prev_sum):
    shape = jax.ShapeDtypeStruct((N * D // 128, 128), jnp.float32)
    return pl.pallas_call(
        _combine_body,
        out_shape=(shape, shape),
    )(p0, p1, prev_sum)


PAIRS_PER_WORKER = B // NW
PAIR_ROWS = PAIRS_PER_WORKER // LANES


@functools.partial(
    pl.kernel,
    out_type=jax.ShapeDtypeStruct((B // LANES, LANES), jnp.float32),
    mesh=_mesh,
    scratch_types=[
        pltpu.VMEM((PAIR_ROWS, LANES), jnp.int32),
        pltpu.VMEM((PAIR_ROWS, LANES), jnp.int32),
        pltpu.VMEM((PAIR_ROWS, LANES, D), jnp.float32),
        pltpu.VMEM((PAIR_ROWS, LANES, D), jnp.float32),
        pltpu.VMEM((PAIR_ROWS, LANES, D), jnp.float32),
        pltpu.VMEM((PAIR_ROWS, LANES), jnp.float32),
        pltpu.SemaphoreType.DMA,
    ],
)
def _score(users_hbm, items_hbm, light_hbm, stats_hbm, out_hbm,
           uidx, iidx, urows, irows, srows, obuf, sem):
    cid = lax.axis_index("c")
    sid = lax.axis_index("s")
    wid = sid * NC + cid
    r0 = wid * PAIR_ROWS

    pltpu.sync_copy(users_hbm.at[pl.ds(r0, PAIR_ROWS)], uidx)
    pltpu.sync_copy(items_hbm.at[pl.ds(r0, PAIR_ROWS)], iidx)
    pltpu.async_copy(light_hbm.at[uidx], urows, sem).wait()
    pltpu.async_copy(light_hbm.at[iidx], irows, sem).wait()
    pltpu.async_copy(stats_hbm.at[uidx], srows, sem).wait()

    def pair(e, _):
        r = e >> 7
        c = e & 127
        g = jnp.sum(urows[r, c, :] * irows[r, c, :]) * (1.0 / 16.0)
        obuf[r, c] = g * srows[r, c, 1] + srows[r, c, 0]
        return ()

    lax.fori_loop(0, PAIRS_PER_WORKER, pair, (), unroll=4)
    pltpu.sync_copy(obuf, out_hbm.at[pl.ds(r0, PAIR_ROWS)])


def kernel(users, items, emb_user, emb_item, edge_src, edge_dst, edge_vals,
           norm_means, norm_stds):
    table = jnp.concatenate([emb_user, emb_item], axis=0)

    pad = EP - E
    src2d = jnp.concatenate(
        [edge_src, jnp.zeros((pad,), jnp.int32)]).reshape(EP // LANES, LANES)
    dst2d = jnp.concatenate(
        [edge_dst, jnp.zeros((pad,), jnp.int32)]).reshape(EP // LANES, LANES)
    vals2d = jnp.concatenate(
        [edge_vals, jnp.zeros((pad,), jnp.float32)]).reshape(EP // LANES, LANES)

    running = table.reshape(N * D // 128, 128)
    for _ in range(N_LAYERS):
        partials = _propagate(src2d, dst2d, vals2d, table)
        tflat, running = _combine(
            partials[0].reshape(N * D // 128, 128),
            partials[1].reshape(N * D // 128, 128),
            running,
        )
        table = tflat.reshape(N, D)

    light = running.reshape(N, D)
    users2d = users.reshape(B // LANES, LANES)
    items2d = (items + NU).reshape(B // LANES, LANES)
    stats = jnp.concatenate(
        [norm_means[:, None], norm_stds[:, None],
         jnp.zeros((NU, D - 2), jnp.float32)], axis=1)

    scores = _score(users2d, items2d, light, stats)
    return scores.reshape(B)


# speedup vs baseline: 19.0082x; 19.0082x over previous
import functools

import jax
import jax.numpy as jnp
from jax import lax
from jax.experimental import pallas as pl
from jax.experimental.pallas import tpu as pltpu
from jax.experimental.pallas import tpu_sc as plsc

NU = 50000
NI = 50000
N = NU + NI
E = 3_200_000
D = 16
B = 16384
N_LAYERS = 3

NC = 2
NS = 16
NW = NC * NS

CHUNK = 1024
ITERS = 100
EDGES_PER_WORKER = CHUNK * ITERS
EP = EDGES_PER_WORKER * NW

NPAD = 100352
NODES_PER_TILE = NPAD // NS

_mesh = plsc.VectorSubcoreMesh(
    core_axis_name="c", subcore_axis_name="s", num_cores=NC, num_subcores=NS
)
_sc_params = pltpu.CompilerParams(use_tc_tiling_on_sc=False, needs_layout_passes=False)


@functools.partial(
    pl.kernel,
    out_type=jax.ShapeDtypeStruct((NC, NPAD, D), jnp.float32),
    mesh=_mesh,
    compiler_params=_sc_params,
    scratch_types=[
        pltpu.VMEM((CHUNK,), jnp.int32),
        pltpu.VMEM((CHUNK,), jnp.int32),
        pltpu.VMEM((CHUNK,), jnp.float32),
        pltpu.VMEM((CHUNK, D), jnp.float32),
        pltpu.VMEM_SHARED((NPAD, D), jnp.float32),
        pltpu.SemaphoreType.DMA,
    ],
)
def _propagate(src_hbm, dst_hbm, vals_hbm, table_hbm, out_hbm,
               sidx, didx, vref, rows, acc, sem):
    cid = lax.axis_index("c")
    sid = lax.axis_index("s")
    wid = sid * NC + cid

    def zrow(g, _):
        rows[g, :] = jnp.zeros((D,), jnp.float32)
        return ()
    lax.fori_loop(0, CHUNK, zrow, ())
    tile_lo = sid * NODES_PER_TILE
    for z in range(NODES_PER_TILE // CHUNK):
        pltpu.sync_copy(rows, acc.at[pl.ds(tile_lo + z * CHUNK, CHUNK)])
    pltpu.sync_copy(
        rows.at[pl.ds(0, NODES_PER_TILE % CHUNK)],
        acc.at[pl.ds(tile_lo + (NODES_PER_TILE // CHUNK) * CHUNK,
                     NODES_PER_TILE % CHUNK)],
    )
    plsc.subcore_barrier()

    edge_base = wid * EDGES_PER_WORKER

    def body(it, _):
        e0 = edge_base + it * CHUNK
        pltpu.sync_copy(src_hbm.at[pl.ds(e0, CHUNK)], sidx)
        pltpu.sync_copy(dst_hbm.at[pl.ds(e0, CHUNK)], didx)
        pltpu.sync_copy(vals_hbm.at[pl.ds(e0, CHUNK)], vref)
        pltpu.async_copy(table_hbm.at[sidx], rows, sem).wait()

        def scale(g, _):
            vals16 = vref[pl.ds(g * 16, 16)]
            base = g * 16
            for j in range(16):
                rows[base + j, :] = rows[base + j, :] * vals16[j]
            return ()

        lax.fori_loop(0, CHUNK // 16, scale, ())
        pltpu.sync_copy(rows, acc.at[didx], add=True)
        return ()

    lax.fori_loop(0, ITERS, body, ())

    plsc.subcore_barrier()
    lo = sid * NODES_PER_TILE
    pltpu.sync_copy(
        acc.at[pl.ds(lo, NODES_PER_TILE)],
        out_hbm.at[cid].at[pl.ds(lo, NODES_PER_TILE)],
    )


def _combine_body(p0_ref, p1_ref, sum_ref, table_out, sum_out):
    t = p0_ref[...] + p1_ref[...]
    table_out[...] = t
    sum_out[...] = sum_ref[...] + t


def _combine(p0, p1, prev_sum):
    shape = jax.ShapeDtypeStruct((NPAD * D // 128, 128), jnp.float32)
    return pl.pallas_call(
        _combine_body,
        out_shape=(shape, shape),
    )(p0, p1, prev_sum)


PAIRS_PER_WORKER = B // NW


@functools.partial(
    pl.kernel,
    out_type=jax.ShapeDtypeStruct((B,), jnp.float32),
    mesh=_mesh,
    compiler_params=_sc_params,
    scratch_types=[
        pltpu.VMEM((PAIRS_PER_WORKER,), jnp.int32),
        pltpu.VMEM((PAIRS_PER_WORKER,), jnp.int32),
        pltpu.VMEM((PAIRS_PER_WORKER, D), jnp.float32),
        pltpu.VMEM((PAIRS_PER_WORKER, D), jnp.float32),
        pltpu.VMEM((PAIRS_PER_WORKER, D), jnp.float32),
        pltpu.VMEM((PAIRS_PER_WORKER,), jnp.float32),
        pltpu.SemaphoreType.DMA,
    ],
)
def _score(users_hbm, items_hbm, light_hbm, stats_hbm, out_hbm,
           uidx, iidx, urows, irows, srows, obuf, sem):
    cid = lax.axis_index("c")
    sid = lax.axis_index("s")
    wid = sid * NC + cid
    p0 = wid * PAIRS_PER_WORKER

    pltpu.sync_copy(users_hbm.at[pl.ds(p0, PAIRS_PER_WORKER)], uidx)
    pltpu.sync_copy(items_hbm.at[pl.ds(p0, PAIRS_PER_WORKER)], iidx)
    pltpu.async_copy(light_hbm.at[uidx], urows, sem).wait()
    pltpu.async_copy(light_hbm.at[iidx], irows, sem).wait()
    pltpu.async_copy(stats_hbm.at[uidx], srows, sem).wait()

    def group(g, _):
        pairs = lax.iota(jnp.int32, 16) + g * 16
        acc = jnp.zeros((16,), jnp.float32)
        for d in range(D):
            col = jnp.full((16,), d, jnp.int32)
            u_d = plsc.load_gather(urows, [pairs, col])
            i_d = plsc.load_gather(irows, [pairs, col])
            acc = acc + u_d * i_d
        means = plsc.load_gather(srows, [pairs, jnp.zeros((16,), jnp.int32)])
        stds = plsc.load_gather(srows, [pairs, jnp.ones((16,), jnp.int32)])
        obuf[pl.ds(g * 16, 16)] = acc * (1.0 / 16.0) * stds + means
        return ()

    lax.fori_loop(0, PAIRS_PER_WORKER // 16, group, ())
    pltpu.sync_copy(obuf, out_hbm.at[pl.ds(p0, PAIRS_PER_WORKER)])


def kernel(users, items, emb_user, emb_item, edge_src, edge_dst, edge_vals,
           norm_means, norm_stds):
    table = jnp.concatenate(
        [emb_user, emb_item,
         jnp.zeros((NPAD - N, D), jnp.float32)], axis=0)

    pad = EP - E
    src_p = jnp.concatenate([edge_src, jnp.zeros((pad,), jnp.int32)])
    dst_p = jnp.concatenate([edge_dst, jnp.zeros((pad,), jnp.int32)])
    vals_p = jnp.concatenate([edge_vals, jnp.zeros((pad,), jnp.float32)])

    running = table.reshape(NPAD * D // 128, 128)
    for _ in range(N_LAYERS):
        partials = _propagate(src_p, dst_p, vals_p, table)
        tflat, running = _combine(
            partials[0].reshape(NPAD * D // 128, 128),
            partials[1].reshape(NPAD * D // 128, 128),
            running,
        )
        table = tflat.reshape(NPAD, D)

    light = running.reshape(NPAD, D)
    items_sh = items + NU
    stats = jnp.concatenate(
        [norm_means[:, None], norm_stds[:, None],
         jnp.zeros((NU, D - 2), jnp.float32)], axis=1)

    return _score(users, items_sh, light, stats)
